# baseline (speedup 1.0000x reference)
import functools

import jax
import jax.numpy as jnp
from jax import lax
from jax.experimental import pallas as pl
from jax.experimental.pallas import tpu as pltpu
from jax.experimental.pallas import tpu_sc as plsc

B = 16384
D = 16
NC = 2
NS = 16
NW = NC * NS
BPW = B // NW


_sc_mesh = plsc.VectorSubcoreMesh(core_axis_name="c", subcore_axis_name="s")


@functools.partial(
    pl.kernel,
    mesh=_sc_mesh,
    compiler_params=pltpu.CompilerParams(use_tc_tiling_on_sc=False),
    out_type=[jax.ShapeDtypeStruct((B, D), jnp.float32)] * 4,
    scratch_types=[
        pltpu.VMEM((BPW,), jnp.int32),
        pltpu.VMEM((BPW,), jnp.int32),
        pltpu.VMEM((BPW, D), jnp.float32),
        pltpu.VMEM((BPW, D), jnp.float32),
        pltpu.VMEM((BPW, D), jnp.float32),
        pltpu.VMEM((BPW, D), jnp.float32),
        pltpu.SemaphoreType.DMA,
    ],
)
def _sc_gather4(x_uid_hbm, x_mid_hbm, t_urc, t_mrc, t_uln, t_mln,
                o_urc, o_mrc, o_uln, o_mln,
                uid_idx, mid_idx, b_urc, b_mrc, b_uln, b_mln, sem):
    wid = lax.axis_index("s") * NC + lax.axis_index("c")
    base = wid * BPW
    pltpu.sync_copy(x_uid_hbm.at[pl.ds(base, BPW)], uid_idx)
    pltpu.sync_copy(x_mid_hbm.at[pl.ds(base, BPW)], mid_idx)
    c1 = pltpu.async_copy(t_urc.at[uid_idx], b_urc, sem)
    c2 = pltpu.async_copy(t_mrc.at[mid_idx], b_mrc, sem)
    c3 = pltpu.async_copy(t_uln.at[uid_idx], b_uln, sem)
    c4 = pltpu.async_copy(t_mln.at[mid_idx], b_mln, sem)
    c1.wait()
    c2.wait()
    c3.wait()
    c4.wait()
    pltpu.sync_copy(b_urc, o_urc.at[pl.ds(base, BPW)])
    pltpu.sync_copy(b_mrc, o_mrc.at[pl.ds(base, BPW)])
    pltpu.sync_copy(b_uln, o_uln.at[pl.ds(base, BPW)])
    pltpu.sync_copy(b_mln, o_mln.at[pl.ds(base, BPW)])


def _mlp_body(urc, mrc, uln, mln, lr,
              W1Trc, b1rc, W2rc, b2rc, W1Tln, b1ln, W2ln, b2ln, y_ref):
    drc = (urc[...] - mrc[...]) * 0.5
    dln = (uln[...] - mln[...]) * 0.5
    h_rc = jnp.maximum(
        jnp.dot(drc, jnp.exp(W1Trc[...]), preferred_element_type=jnp.float32)
        + b1rc[...], 0.0)
    a_rc = jnp.sum(h_rc * jnp.exp(W2rc[...]), axis=1) + b2rc[0]
    h_ln = jnp.maximum(
        jnp.dot(dln, jnp.exp(W1Tln[...]), preferred_element_type=jnp.float32)
        + b1ln[...], 0.0)
    a_ln = jnp.sum(h_ln * jnp.exp(W2ln[...]), axis=1) + b2ln[0]
    s_rc = 1.0 / (1.0 + jnp.exp(-a_rc))
    s_ln = 1.0 / (1.0 + jnp.exp(-a_ln))
    r = lr[...]
    y_ref[...] = s_rc * (1.0 - r) + s_ln * r


_BBLK = 2048


def _mlp(urc, mrc, uln, mln, lr, W1rc, b1rc, W2rc, b2rc, W1ln, b1ln, W2ln, b2ln):
    bspec = pl.BlockSpec((_BBLK, D), lambda i: (i, 0))
    rspec = pl.BlockSpec((_BBLK,), lambda i: (i,))
    w1spec = pl.BlockSpec((D, 32), lambda i: (0, 0))
    w2spec = pl.BlockSpec((1, 32), lambda i: (0, 0))
    sspec = pl.BlockSpec(memory_space=pltpu.SMEM)
    return pl.pallas_call(
        _mlp_body,
        grid=(B // _BBLK,),
        in_specs=[bspec] * 4 + [rspec,
                                w1spec, w2spec, w2spec, sspec,
                                w1spec, w2spec, w2spec, sspec],
        out_specs=rspec,
        out_shape=jax.ShapeDtypeStruct((B,), jnp.float32),
    )(urc, mrc, uln, mln, lr,
      W1rc.T, b1rc.reshape(1, -1), W2rc, b2rc,
      W1ln.T, b1ln.reshape(1, -1), W2ln, b2ln)


def kernel(x_uid, x_mid, ln_ratio, uid_emb_rc, mid_emb_rc, uid_emb_ln,
           mid_emb_ln, W1_rc, b1_rc, W2_rc, b2_rc, W1_ln, b1_ln, W2_ln, b2_ln):
    urc, mrc, uln, mln = _sc_gather4(
        x_uid.astype(jnp.int32), x_mid.astype(jnp.int32),
        uid_emb_rc, mid_emb_rc, uid_emb_ln, mid_emb_ln)
    return _mlp(urc, mrc, uln, mln, ln_ratio,
                W1_rc, b1_rc, W2_rc, b2_rc, W1_ln, b1_ln, W2_ln, b2_ln)
